# Initial kernel scaffold; baseline (speedup 1.0000x reference)
import jax
import jax.numpy as jnp
from jax.experimental import pallas as pl


def kernel(x, W_enc, b_enc, W_dec, b_dec):
    raise NotImplementedError("write your pallas kernel here")


# speedup vs baseline: 10.6805x; 10.6805x over previous
import functools

import jax
import jax.numpy as jnp
from jax.experimental import pallas as pl
from jax.experimental.pallas import tpu as pltpu

N_TOKENS = 2048
D_IN = 2048
D_SAE = 16384
K = 64

INT32_MIN = -(2**31)


def _encode_body(x_ref, w_ref, b_ref, z_ref):
    z = jax.lax.dot_general(
        x_ref[...], w_ref[...],
        (((1,), (1,)), ((), ())),
        preferred_element_type=jnp.float32,
    )
    z_ref[...] = z + b_ref[...]


def _select_body(z_ref, h_ref):
    z = z_ref[...]
    u = jax.lax.bitcast_convert_type(z, jnp.int32)
    key = jnp.where(u >= 0, u, INT32_MIN - u)

    br = z.shape[0]

    def body(i, carry):
        lo, hi = carry
        mid = (lo >> 1) + (hi >> 1) + (lo & hi & 1)
        cnt = jnp.sum((key >= mid).astype(jnp.int32), axis=1, keepdims=True)
        ge = cnt >= K
        return jnp.where(ge, mid, lo), jnp.where(ge, hi, mid)

    lo0 = jnp.full((br, 1), INT32_MIN, dtype=jnp.int32)
    hi0 = jnp.full((br, 1), 2**31 - 1, dtype=jnp.int32)
    t, _ = jax.lax.fori_loop(0, 32, body, (lo0, hi0))
    mask = key >= t
    h_ref[...] = jnp.where(mask, jnp.maximum(z, 0.0), 0.0)


def _decode_body(h_ref, w_ref, b_ref, out_ref):
    k = pl.program_id(0)

    @pl.when(k == 0)
    def _():
        out_ref[...] = jnp.broadcast_to(b_ref[...], out_ref.shape)

    out_ref[...] += jax.lax.dot_general(
        h_ref[...].astype(jnp.bfloat16), w_ref[...],
        (((1,), (1,)), ((), ())),
        preferred_element_type=jnp.float32,
    )


@jax.jit
def kernel(x, W_enc, b_enc, W_dec, b_dec):
    n, d_in = x.shape
    d_sae = W_enc.shape[0]

    BN = 512
    z = pl.pallas_call(
        _encode_body,
        grid=(d_sae // BN,),
        in_specs=[
            pl.BlockSpec((n, d_in), lambda j: (0, 0)),
            pl.BlockSpec((BN, d_in), lambda j: (j, 0)),
            pl.BlockSpec((1, BN), lambda j: (0, j)),
        ],
        out_specs=pl.BlockSpec((n, BN), lambda j: (0, j)),
        out_shape=jax.ShapeDtypeStruct((n, d_sae), jnp.float32),
    )(x, W_enc, b_enc.reshape(1, d_sae))

    BR = 128
    hidden = pl.pallas_call(
        _select_body,
        grid=(n // BR,),
        in_specs=[pl.BlockSpec((BR, d_sae), lambda i: (i, 0))],
        out_specs=pl.BlockSpec((BR, d_sae), lambda i: (i, 0)),
        out_shape=jax.ShapeDtypeStruct((n, d_sae), jnp.float32),
    )(z)

    BK = 1024
    W_dec_bf = W_dec.astype(jnp.bfloat16)
    recon = pl.pallas_call(
        _decode_body,
        grid=(d_sae // BK,),
        in_specs=[
            pl.BlockSpec((n, BK), lambda k: (0, k)),
            pl.BlockSpec((d_in, BK), lambda k: (0, k)),
            pl.BlockSpec((1, d_in), lambda k: (0, 0)),
        ],
        out_specs=pl.BlockSpec((n, d_in), lambda k: (0, 0)),
        out_shape=jax.ShapeDtypeStruct((n, d_in), jnp.float32),
        compiler_params=pltpu.CompilerParams(
            dimension_semantics=("arbitrary",),
        ),
    )(hidden, W_dec_bf, b_dec.reshape(1, d_in))

    return (hidden, recon)
